# baseline (speedup 1.0000x reference)
import jax
import jax.numpy as jnp
import numpy as np
from jax import lax
from jax.experimental import pallas as pl
from jax.experimental.pallas import tpu as pltpu
from jax.experimental.pallas import tpu_sc as plsc

B = 16
N = 100000
H = W = 512
THRESH = 1.0 - 0.03

TBL_ROWS = H + 8
TBL_WORDS = TBL_ROWS * 32
SENTINEL_BASE = H * W

NUM_CORES = 2
NUM_SUBCORES = 16
NUM_WORKERS = NUM_CORES * NUM_SUBCORES
PW = (B * N) // NUM_WORKERS
LANES = 16


IDX_COLS = 400
IDX_ROWS = (B * N * 2) // IDX_COLS
IDX_BLOCK_ROWS = 800
PAIRS = IDX_COLS // 2


def _idx_body(q_ref, m_ref, out_ref):
    q = q_ref[...]
    v = jnp.round(((q + 1.0) * 512.0 - 1.0) / 2.0)
    valid = (v >= 0.0) & (v <= 511.0)
    vc = jnp.clip(v, 0.0, 511.0)
    vp = jnp.where(valid, vc, 1e7)
    a1 = jnp.floor(vp * 0.25)
    a0 = vp - 4.0 * a1
    vsplit = jnp.concatenate([a1, a0], axis=1)
    c_f = jnp.dot(vsplit, m_ref[...], preferred_element_type=jnp.float32)
    spread = lax.broadcasted_iota(jnp.int32, c_f.shape, 1)
    c = jnp.where(
        c_f > float(SENTINEL_BASE) - 0.5,
        SENTINEL_BASE + (spread & 4095),
        c_f.astype(jnp.int32),
    )
    out_ref[...] = c


def _pair_matrix() -> np.ndarray:
    m = np.zeros((2 * IDX_COLS, PAIRS), np.float32)
    k = np.arange(PAIRS)
    m[2 * k, k] = 4.0
    m[2 * k + 1, k] = 2048.0
    m[IDX_COLS + 2 * k, k] = 1.0
    m[IDX_COLS + 2 * k + 1, k] = 512.0
    return m


def _compute_indices(qflat):
    m = jnp.asarray(_pair_matrix())
    grid = IDX_ROWS // IDX_BLOCK_ROWS
    return pl.pallas_call(
        _idx_body,
        grid=(grid,),
        in_specs=[
            pl.BlockSpec((IDX_BLOCK_ROWS, IDX_COLS), lambda i: (i, 0)),
            pl.BlockSpec((2 * IDX_COLS, PAIRS), lambda i: (0, 0)),
        ],
        out_specs=pl.BlockSpec((IDX_BLOCK_ROWS, PAIRS), lambda i: (i, 0)),
        out_shape=jax.ShapeDtypeStruct((IDX_ROWS, PAIRS), jnp.int32),
    )(qflat, m)


def _tbl_body(img_ref, p_ref, tbl_ref):
    t = (img_ref[0] < THRESH).astype(jnp.float32)
    w = jnp.dot(t, p_ref[...], preferred_element_type=jnp.float32)
    wi = w.astype(jnp.int32)
    sent = jnp.full((8, 32), 65535, jnp.int32)
    tbl_ref[0] = jnp.concatenate([wi, sent], axis=0)


def _pack_matrix() -> np.ndarray:
    p = np.zeros((512, 32), np.float32)
    x = np.arange(512)
    p[x, x >> 4] = (1 << (x & 15)).astype(np.float32)
    return p


def _compute_table(img):
    p = jnp.asarray(_pack_matrix())
    return pl.pallas_call(
        _tbl_body,
        grid=(B,),
        in_specs=[
            pl.BlockSpec((1, H, W), lambda b: (b, 0, 0)),
            pl.BlockSpec((512, 32), lambda b: (0, 0)),
        ],
        out_specs=pl.BlockSpec((1, TBL_ROWS, 32), lambda b: (b, 0, 0)),
        out_shape=jax.ShapeDtypeStruct((B, TBL_ROWS, 32), jnp.int32),
    )(img, p)


UNROLL = 5
STEPS = PW // (LANES * UNROLL)


def _sc_body(tbl_hbm, idx_hbm, out_hbm, tbl_v, idx_v, res_v):
    wid = lax.axis_index("s") * NUM_CORES + lax.axis_index("c")
    img = wid // 2
    base = wid * PW
    pltpu.sync_copy(tbl_hbm.at[img], tbl_v)
    pltpu.sync_copy(idx_hbm.at[pl.ds(base, PW)], idx_v)

    def step(i, carry):
        for u in range(UNROLL):
            s = (i * UNROLL + u) * LANES
            c = idx_v[pl.ds(s, LANES)]
            word = plsc.load_gather(tbl_v, [lax.shift_right_logical(c, 4)])
            bit = lax.bitwise_and(c, 15)
            r = lax.bitwise_and(lax.shift_right_logical(word, bit), 1)
            res_v[pl.ds(s, LANES)] = r
        return carry

    lax.fori_loop(0, STEPS, step, 0)
    pltpu.sync_copy(res_v, out_hbm.at[pl.ds(base, PW)])


def _sc_gather(tbl, idx):
    mesh = plsc.VectorSubcoreMesh(core_axis_name="c", subcore_axis_name="s")
    f = pl.kernel(
        _sc_body,
        out_type=jax.ShapeDtypeStruct((B * N,), jnp.int32),
        mesh=mesh,
        scratch_types=[
            pltpu.VMEM((TBL_WORDS,), jnp.int32),
            pltpu.VMEM((PW,), jnp.int32),
            pltpu.VMEM((PW,), jnp.int32),
        ],
        compiler_params=pltpu.CompilerParams(needs_layout_passes=False),
    )
    return f(tbl, idx)


def kernel(query_pts, reference):
    img = reference.reshape(B, H, W)
    qflat = query_pts.reshape(IDX_ROWS, IDX_COLS)
    idxc = _compute_indices(qflat)
    tbl = _compute_table(img)
    res = _sc_gather(tbl.reshape(B, TBL_WORDS), idxc.reshape(B * N))
    return res.reshape(B, N).astype(jnp.bool_)


# speedup vs baseline: 11.2706x; 11.2706x over previous
import jax
import jax.numpy as jnp
import numpy as np
from jax import lax
from jax.experimental import pallas as pl
from jax.experimental.pallas import tpu as pltpu
from jax.experimental.pallas import tpu_sc as plsc

B = 16
N = 100000
H = W = 512
THRESH = 1.0 - 0.03

TBL_ROWS = 40
TBL_WORDS = TBL_ROWS * 512
SENTINEL_BASE = H * W

NUM_CORES = 2
NUM_SUBCORES = 16
NUM_WORKERS = NUM_CORES * NUM_SUBCORES
PW = (B * N) // NUM_WORKERS
LANES = 16

PL_ROWS = 1000
PL_COLS = 1600
PL_BLOCK_ROWS = 200


def _idx_body(x_ref, y_ref, out_ref):
    x = x_ref[...]
    y = y_ref[...]
    vx = jnp.round(((x + 1.0) * 512.0 - 1.0) / 2.0)
    vy = jnp.round(((y + 1.0) * 512.0 - 1.0) / 2.0)
    valid = (vx >= 0.0) & (vx <= 511.0) & (vy >= 0.0) & (vy <= 511.0)
    ix = jnp.clip(vx, 0.0, 511.0).astype(jnp.int32)
    iy = jnp.clip(vy, 0.0, 511.0).astype(jnp.int32)
    c = (
        lax.shift_left(lax.shift_right_logical(iy, 4), 13)
        | lax.shift_left(ix, 4)
        | (iy & 15)
    )
    spread = lax.broadcasted_iota(jnp.int32, c.shape, 1) & 4095
    out_ref[...] = jnp.where(valid, c, SENTINEL_BASE + spread)


def _compute_addresses(xs, ys):
    grid = PL_ROWS // PL_BLOCK_ROWS
    return pl.pallas_call(
        _idx_body,
        grid=(grid,),
        in_specs=[
            pl.BlockSpec((PL_BLOCK_ROWS, PL_COLS), lambda i: (i, 0)),
            pl.BlockSpec((PL_BLOCK_ROWS, PL_COLS), lambda i: (i, 0)),
        ],
        out_specs=pl.BlockSpec((PL_BLOCK_ROWS, PL_COLS), lambda i: (i, 0)),
        out_shape=jax.ShapeDtypeStruct((PL_ROWS, PL_COLS), jnp.int32),
    )(xs, ys)


def _tbl_body(img_ref, p_ref, tbl_ref):
    t = (img_ref[0] < THRESH).astype(jnp.float32)
    w = jnp.dot(p_ref[...], t, preferred_element_type=jnp.float32)
    wi = w.astype(jnp.int32)
    sent = jnp.full((8, 512), 65535, jnp.int32)
    tbl_ref[0] = jnp.concatenate([wi, sent], axis=0)


def _pack_matrix() -> np.ndarray:
    p = np.zeros((32, 512), np.float32)
    iy = np.arange(512)
    p[iy >> 4, iy] = (1 << (iy & 15)).astype(np.float32)
    return p


def _compute_table(img):
    p = jnp.asarray(_pack_matrix())
    return pl.pallas_call(
        _tbl_body,
        grid=(B,),
        in_specs=[
            pl.BlockSpec((1, H, W), lambda b: (b, 0, 0)),
            pl.BlockSpec((32, 512), lambda b: (0, 0)),
        ],
        out_specs=pl.BlockSpec((1, TBL_ROWS, 512), lambda b: (b, 0, 0)),
        out_shape=jax.ShapeDtypeStruct((B, TBL_ROWS, 512), jnp.int32),
    )(img, p)


UNROLL = 5
STEPS = PW // (LANES * UNROLL)


def _sc_body(tbl_hbm, idx_hbm, out_hbm, tbl_v, idx_v, res_v):
    wid = lax.axis_index("s") * NUM_CORES + lax.axis_index("c")
    img = wid // 2
    base = wid * PW
    pltpu.sync_copy(tbl_hbm.at[img], tbl_v)
    pltpu.sync_copy(idx_hbm.at[pl.ds(base, PW)], idx_v)

    def step(i, carry):
        for u in range(UNROLL):
            s = (i * UNROLL + u) * LANES
            c = idx_v[pl.ds(s, LANES)]
            word = plsc.load_gather(tbl_v, [lax.shift_right_logical(c, 4)])
            bit = lax.bitwise_and(c, 15)
            r = lax.bitwise_and(lax.shift_right_logical(word, bit), 1)
            res_v[pl.ds(s, LANES)] = r
        return carry

    lax.fori_loop(0, STEPS, step, 0)
    pltpu.sync_copy(res_v, out_hbm.at[pl.ds(base, PW)])


def _sc_gather(tbl, idx):
    mesh = plsc.VectorSubcoreMesh(core_axis_name="c", subcore_axis_name="s")
    f = pl.kernel(
        _sc_body,
        out_type=jax.ShapeDtypeStruct((B * N,), jnp.int32),
        mesh=mesh,
        scratch_types=[
            pltpu.VMEM((TBL_WORDS,), jnp.int32),
            pltpu.VMEM((PW,), jnp.int32),
            pltpu.VMEM((PW,), jnp.int32),
        ],
        compiler_params=pltpu.CompilerParams(needs_layout_passes=False),
    )
    return f(tbl, idx)


def kernel(query_pts, reference):
    img = reference.reshape(B, H, W)
    xs = query_pts[:, :, 0].reshape(PL_ROWS, PL_COLS)
    ys = query_pts[:, :, 1].reshape(PL_ROWS, PL_COLS)
    idxc = _compute_addresses(xs, ys)
    tbl = _compute_table(img)
    res = _sc_gather(tbl.reshape(B, TBL_WORDS), idxc.reshape(B * N))
    return res.reshape(B, N).astype(jnp.bool_)


# speedup vs baseline: 18.1765x; 1.6127x over previous
import jax
import jax.numpy as jnp
import numpy as np
from jax import lax
from jax.experimental import pallas as pl
from jax.experimental.pallas import tpu as pltpu
from jax.experimental.pallas import tpu_sc as plsc

B = 16
N = 100000
H = W = 512
THRESH = 1.0 - 0.03

TBL_ROWS = 40
TBL_WORDS = TBL_ROWS * 512
SENTINEL_BASE = H * W

NUM_CORES = 2
NUM_SUBCORES = 16
NUM_WORKERS = NUM_CORES * NUM_SUBCORES
PW = (B * N) // NUM_WORKERS
LANES = 16

PL_ROWS = 1000
PL_COLS = 1600
PL_BLOCK_ROWS = 200


def _idx_body(q_ref, out_ref):
    x = q_ref[0]
    y = q_ref[1]
    vx = jnp.round(((x + 1.0) * 512.0 - 1.0) / 2.0)
    vy = jnp.round(((y + 1.0) * 512.0 - 1.0) / 2.0)
    valid = (vx >= 0.0) & (vx <= 511.0) & (vy >= 0.0) & (vy <= 511.0)
    ix = jnp.clip(vx, 0.0, 511.0).astype(jnp.int32)
    iy = jnp.clip(vy, 0.0, 511.0).astype(jnp.int32)
    c = (
        lax.shift_left(lax.shift_right_logical(iy, 4), 13)
        | lax.shift_left(ix, 4)
        | (iy & 15)
    )
    spread = lax.broadcasted_iota(jnp.int32, c.shape, 1) & 4095
    out_ref[...] = jnp.where(valid, c, SENTINEL_BASE + spread)


def _compute_addresses(qp):
    grid = PL_ROWS // PL_BLOCK_ROWS
    return pl.pallas_call(
        _idx_body,
        grid=(grid,),
        in_specs=[
            pl.BlockSpec((2, PL_BLOCK_ROWS, PL_COLS), lambda i: (0, i, 0)),
        ],
        out_specs=pl.BlockSpec((PL_BLOCK_ROWS, PL_COLS), lambda i: (i, 0)),
        out_shape=jax.ShapeDtypeStruct((PL_ROWS, PL_COLS), jnp.int32),
    )(qp)


def _tbl_body(img_ref, p_ref, tbl_ref):
    t = (img_ref[0] < THRESH).astype(jnp.float32)
    w = jnp.dot(p_ref[...], t, preferred_element_type=jnp.float32)
    wi = w.astype(jnp.int32)
    sent = jnp.full((8, 512), 65535, jnp.int32)
    tbl_ref[0] = jnp.concatenate([wi, sent], axis=0)


def _pack_matrix() -> np.ndarray:
    p = np.zeros((32, 512), np.float32)
    iy = np.arange(512)
    p[iy >> 4, iy] = (1 << (iy & 15)).astype(np.float32)
    return p


def _compute_table(img):
    p = jnp.asarray(_pack_matrix())
    return pl.pallas_call(
        _tbl_body,
        grid=(B,),
        in_specs=[
            pl.BlockSpec((1, H, W), lambda b: (b, 0, 0)),
            pl.BlockSpec((32, 512), lambda b: (0, 0)),
        ],
        out_specs=pl.BlockSpec((1, TBL_ROWS, 512), lambda b: (b, 0, 0)),
        out_shape=jax.ShapeDtypeStruct((B, TBL_ROWS, 512), jnp.int32),
    )(img, p)


UNROLL = 5
STEPS = PW // (LANES * UNROLL)


def _sc_body(tbl_hbm, idx_hbm, out_hbm, tbl_v, idx_v, res_v):
    wid = lax.axis_index("s") * NUM_CORES + lax.axis_index("c")
    img = wid // 2
    base = wid * PW
    pltpu.sync_copy(tbl_hbm.at[img], tbl_v)
    pltpu.sync_copy(idx_hbm.at[pl.ds(base, PW)], idx_v)

    def step(i, carry):
        for u in range(UNROLL):
            s = (i * UNROLL + u) * LANES
            c = idx_v[pl.ds(s, LANES)]
            word = plsc.load_gather(tbl_v, [lax.shift_right_logical(c, 4)])
            bit = lax.bitwise_and(c, 15)
            r = lax.bitwise_and(lax.shift_right_logical(word, bit), 1)
            res_v[pl.ds(s, LANES)] = r
        return carry

    lax.fori_loop(0, STEPS, step, 0)
    pltpu.sync_copy(res_v, out_hbm.at[pl.ds(base, PW)])


def _sc_gather(tbl, idx):
    mesh = plsc.VectorSubcoreMesh(core_axis_name="c", subcore_axis_name="s")
    f = pl.kernel(
        _sc_body,
        out_type=jax.ShapeDtypeStruct((B * N,), jnp.int32),
        mesh=mesh,
        scratch_types=[
            pltpu.VMEM((TBL_WORDS,), jnp.int32),
            pltpu.VMEM((PW,), jnp.int32),
            pltpu.VMEM((PW,), jnp.int32),
        ],
        compiler_params=pltpu.CompilerParams(needs_layout_passes=False),
    )
    return f(tbl, idx)


def kernel(query_pts, reference):
    img = reference.reshape(B, H, W)
    qp = jnp.transpose(query_pts, (2, 0, 1)).reshape(2, PL_ROWS, PL_COLS)
    idxc = _compute_addresses(qp)
    tbl = _compute_table(img)
    res = _sc_gather(tbl.reshape(B, TBL_WORDS), idxc.reshape(B * N))
    return res.reshape(B, N).astype(jnp.bool_)
